# Initial kernel scaffold; baseline (speedup 1.0000x reference)
import jax
import jax.numpy as jnp
from jax.experimental import pallas as pl


def kernel(pred, target, acc_sum):
    raise NotImplementedError("write your pallas kernel here")


# speedup vs baseline: 102.8282x; 102.8282x over previous
import functools

import jax
import jax.numpy as jnp
from jax import lax
from jax.experimental import pallas as pl
from jax.experimental.pallas import tpu as pltpu
from jax.experimental.pallas import tpu_sc as plsc

_BINS = 30
_MMT = 0.6
_BATCH = 16384
_CLASS = 64

_NC = 2
_NS = 16
_NW = _NC * _NS
_L = 16

_ROWS_PER_TILE = _BATCH // _NW
_WORDS_PER_TILE = _ROWS_PER_TILE * _CLASS
_TBL = _CLASS * _BINS


def _hist_body(pred_hbm, tgt_hbm, cnt_hbm, bsum_hbm, pred_v, tgt_v, cnt_v, bsum_v):
    wid = lax.axis_index("s") * _NC + lax.axis_index("c")
    base = wid * _WORDS_PER_TILE

    pltpu.sync_copy(pred_hbm.at[pl.ds(base, _WORDS_PER_TILE)], pred_v)
    pltpu.sync_copy(tgt_hbm.at[pl.ds(base, _WORDS_PER_TILE)], tgt_v)

    zeros = jnp.zeros((_L,), jnp.float32)

    def zero_body(i, carry):
        cnt_v[pl.ds(i * _L, _L)] = zeros
        bsum_v[pl.ds(i * _L, _L)] = zeros
        return carry

    lax.fori_loop(0, _TBL // _L, zero_body, 0)

    lane = lax.iota(jnp.int32, _L)
    ones = jnp.ones((_L,), jnp.float32)
    base_idx = [(k * _L + lane) * _BINS for k in range(_CLASS // _L)]

    def row_body(r, carry):
        off = r * _CLASS
        for k in range(_CLASS // _L):
            p = pred_v[pl.ds(off + k * _L, _L)]
            t = tgt_v[pl.ds(off + k * _L, _L)]
            q = jnp.where(t > 0, -p, p)
            u = jnp.exp(-jnp.abs(q))
            z = u / (u + 2.0)
            z2 = z * z
            l1p = (2.0 * z) * (1.0 + z2 * (1.0 / 3.0 + z2 * (0.2 + z2 * (1.0 / 7.0 + z2 * (1.0 / 9.0)))))
            bce = jnp.maximum(q, 0.0) + l1p
            eq = jnp.where(q >= 0.0, u, 1.0 / u)
            g = 1.0 / (1.0 + eq)
            b = jnp.minimum((g * float(_BINS)).astype(jnp.int32), _BINS - 1)
            idx = base_idx[k] + b
            plsc.addupdate_scatter(cnt_v, [idx], ones)
            plsc.addupdate_scatter(bsum_v, [idx], bce)
        return carry

    lax.fori_loop(0, _ROWS_PER_TILE, row_body, 0)

    tbase = wid * _TBL
    pltpu.sync_copy(cnt_v, cnt_hbm.at[pl.ds(tbase, _TBL)])
    pltpu.sync_copy(bsum_v, bsum_hbm.at[pl.ds(tbase, _TBL)])


_hist = pl.kernel(
    _hist_body,
    out_type=(
        jax.ShapeDtypeStruct((_NW * _TBL,), jnp.float32),
        jax.ShapeDtypeStruct((_NW * _TBL,), jnp.float32),
    ),
    mesh=plsc.VectorSubcoreMesh(core_axis_name="c", subcore_axis_name="s"),
    compiler_params=pltpu.CompilerParams(needs_layout_passes=False),
    scratch_types=[
        pltpu.VMEM((_WORDS_PER_TILE,), jnp.float32),
        pltpu.VMEM((_WORDS_PER_TILE,), jnp.int32),
        pltpu.VMEM((_TBL,), jnp.float32),
        pltpu.VMEM((_TBL,), jnp.float32),
    ],
)


def _epi_body(cnt_ref, bsum_ref, acc_ref, out_ref):
    cnt = jnp.sum(cnt_ref[...], axis=0)
    bsum = jnp.sum(bsum_ref[...], axis=0)
    acc = acc_ref[...]
    ne = cnt > 0.0
    accn = jnp.where(ne, _MMT * acc + (1.0 - _MMT) * cnt, acc)
    contrib = jnp.where(ne, bsum / jnp.where(ne, accn, 1.0), 0.0)
    n = jnp.sum(ne.astype(jnp.float32), axis=1)
    csum = jnp.sum(contrib, axis=1)
    n = jnp.where(n > 0.0, n, 1.0)
    loss = jnp.sum(csum / n) * (1.0 / _CLASS)
    out_ref[...] = loss[None, None]


_epilogue = pl.pallas_call(
    _epi_body,
    out_shape=jax.ShapeDtypeStruct((1, 1), jnp.float32),
)


def kernel(pred, target, acc_sum):
    cnt, bsum = _hist(pred.reshape(-1), target.reshape(-1))
    loss2d = _epilogue(
        cnt.reshape(_NW, _CLASS, _BINS),
        bsum.reshape(_NW, _CLASS, _BINS),
        acc_sum,
    )
    return loss2d[0, 0]


# speedup vs baseline: 107.1618x; 1.0421x over previous
import functools

import jax
import jax.numpy as jnp
from jax import lax
from jax.experimental import pallas as pl
from jax.experimental.pallas import tpu as pltpu
from jax.experimental.pallas import tpu_sc as plsc

_BINS = 30
_MMT = 0.6
_BATCH = 16384
_CLASS = 64

_NC = 2
_NS = 16
_NW = _NC * _NS
_L = 16

_ROWS_PER_TILE = _BATCH // _NW
_WORDS_PER_TILE = _ROWS_PER_TILE * _CLASS
_TBL = _CLASS * _BINS


def _hist_body(pred_hbm, tgt_hbm, cnt_hbm, bsum_hbm, pred_v, tgt_v, cnt_v, bsum_v):
    wid = lax.axis_index("s") * _NC + lax.axis_index("c")
    base = wid * _WORDS_PER_TILE

    pltpu.sync_copy(pred_hbm.at[pl.ds(base, _WORDS_PER_TILE)], pred_v)
    pltpu.sync_copy(tgt_hbm.at[pl.ds(base, _WORDS_PER_TILE)], tgt_v)

    zeros = jnp.zeros((_L,), jnp.float32)

    def zero_body(i, carry):
        cnt_v[pl.ds(i * _L, _L)] = zeros
        bsum_v[pl.ds(i * _L, _L)] = zeros
        return carry

    lax.fori_loop(0, _TBL // _L, zero_body, 0)

    lane = lax.iota(jnp.int32, _L)
    ones = jnp.ones((_L,), jnp.float32)
    base_idx = [(k * _L + lane) * _BINS for k in range(_CLASS // _L)]

    c = (9.083787e-08, 0.99999146, -0.49980116, 0.33133401, -0.23919072,
         0.1647835, -0.09231377, 0.03441859, -0.00607488)

    def row_body(r, carry):
        off = r * _CLASS
        for k in range(_CLASS // _L):
            p = pred_v[pl.ds(off + k * _L, _L)]
            t = tgt_v[pl.ds(off + k * _L, _L)]
            q = jnp.where(t > 0, -p, p)
            u = jnp.exp(-jnp.abs(p))
            l1p = c[8]
            for j in range(7, -1, -1):
                l1p = l1p * u + c[j]
            bce = jnp.maximum(q, 0.0) + l1p
            r1 = 1.0 / (1.0 + u)
            g = jnp.where(q >= 0.0, r1, u * r1)
            b = jnp.minimum((g * float(_BINS)).astype(jnp.int32), _BINS - 1)
            idx = base_idx[k] + b
            plsc.addupdate_scatter(cnt_v, [idx], ones)
            plsc.addupdate_scatter(bsum_v, [idx], bce)
        return carry

    lax.fori_loop(0, _ROWS_PER_TILE, row_body, 0, unroll=2)

    tbase = wid * _TBL
    pltpu.sync_copy(cnt_v, cnt_hbm.at[pl.ds(tbase, _TBL)])
    pltpu.sync_copy(bsum_v, bsum_hbm.at[pl.ds(tbase, _TBL)])


_hist = pl.kernel(
    _hist_body,
    out_type=(
        jax.ShapeDtypeStruct((_NW * _TBL,), jnp.float32),
        jax.ShapeDtypeStruct((_NW * _TBL,), jnp.float32),
    ),
    mesh=plsc.VectorSubcoreMesh(core_axis_name="c", subcore_axis_name="s"),
    compiler_params=pltpu.CompilerParams(needs_layout_passes=False),
    scratch_types=[
        pltpu.VMEM((_WORDS_PER_TILE,), jnp.float32),
        pltpu.VMEM((_WORDS_PER_TILE,), jnp.int32),
        pltpu.VMEM((_TBL,), jnp.float32),
        pltpu.VMEM((_TBL,), jnp.float32),
    ],
)


def _epi_body(cnt_ref, bsum_ref, acc_ref, out_ref):
    cnt = jnp.sum(cnt_ref[...], axis=0)
    bsum = jnp.sum(bsum_ref[...], axis=0)
    acc = acc_ref[...]
    ne = cnt > 0.0
    accn = jnp.where(ne, _MMT * acc + (1.0 - _MMT) * cnt, acc)
    contrib = jnp.where(ne, bsum / jnp.where(ne, accn, 1.0), 0.0)
    n = jnp.sum(ne.astype(jnp.float32), axis=1)
    csum = jnp.sum(contrib, axis=1)
    n = jnp.where(n > 0.0, n, 1.0)
    loss = jnp.sum(csum / n) * (1.0 / _CLASS)
    out_ref[...] = loss[None, None]


_epilogue = pl.pallas_call(
    _epi_body,
    out_shape=jax.ShapeDtypeStruct((1, 1), jnp.float32),
)


def kernel(pred, target, acc_sum):
    cnt, bsum = _hist(pred.reshape(-1), target.reshape(-1))
    loss2d = _epilogue(
        cnt.reshape(_NW, _CLASS, _BINS),
        bsum.reshape(_NW, _CLASS, _BINS),
        acc_sum,
    )
    return loss2d[0, 0]


# speedup vs baseline: 183.1602x; 1.7092x over previous
import functools

import jax
import jax.numpy as jnp
from jax import lax
from jax.experimental import pallas as pl
from jax.experimental.pallas import tpu as pltpu
from jax.experimental.pallas import tpu_sc as plsc

_BINS = 30
_MMT = 0.6
_BATCH = 16384
_CLASS = 64

_NC = 2
_NS = 16
_NW = _NC * _NS
_L = 16

_ROWS_PER_TILE = _BATCH // _NW
_WORDS_PER_TILE = _ROWS_PER_TILE * _CLASS
_TBL = _CLASS * _BINS


def _hist_body(pred_hbm, tgt_hbm, cnt_hbm, bsum_hbm, pred_v, tgt_v, cnt_v, bsum_v):
    wid = lax.axis_index("s") * _NC + lax.axis_index("c")
    base = wid * _WORDS_PER_TILE

    pltpu.sync_copy(pred_hbm.at[pl.ds(base, _WORDS_PER_TILE)], pred_v)
    pltpu.sync_copy(tgt_hbm.at[pl.ds(base, _WORDS_PER_TILE)], tgt_v)

    zeros = jnp.zeros((_L,), jnp.float32)

    def zero_body(i, carry):
        cnt_v[pl.ds(i * _L, _L)] = zeros
        bsum_v[pl.ds(i * _L, _L)] = zeros
        return carry

    lax.fori_loop(0, _TBL // _L, zero_body, 0)

    lane = lax.iota(jnp.int32, _L)
    ones = jnp.ones((_L,), jnp.float32)
    base_idx = [(k * _L + lane) * _BINS for k in range(_CLASS // _L)]

    c = (2.211703e-05, 0.99901044, -0.48915684, 0.28330433, -0.13011941,
         0.030102625)

    _RPI = 2
    _KV = _RPI * (_CLASS // _L)

    def row_body(it, carry):
        off = it * (_RPI * _CLASS)
        ps = [pred_v[pl.ds(off + k * _L, _L)] for k in range(_KV)]
        ts = [tgt_v[pl.ds(off + k * _L, _L)] for k in range(_KV)]
        qs = [jnp.where(t > 0, -p, p) for p, t in zip(ps, ts)]
        us = [jnp.exp(-jnp.abs(p)) for p in ps]
        for k in range(_KV):
            q, u = qs[k], us[k]
            l1p = c[5]
            for j in range(4, -1, -1):
                l1p = l1p * u + c[j]
            bce = jnp.maximum(q, 0.0) + l1p
            r1 = 1.0 / (1.0 + u)
            g = jnp.where(q >= 0.0, r1, u * r1)
            b = jnp.minimum((g * float(_BINS)).astype(jnp.int32), _BINS - 1)
            idx = base_idx[k % (_CLASS // _L)] + b
            plsc.addupdate_scatter(cnt_v, [idx], ones)
            plsc.addupdate_scatter(bsum_v, [idx], bce)
        return carry

    lax.fori_loop(0, _ROWS_PER_TILE // _RPI, row_body, 0, unroll=2)

    tbase = wid * _TBL
    pltpu.sync_copy(cnt_v, cnt_hbm.at[pl.ds(tbase, _TBL)])
    pltpu.sync_copy(bsum_v, bsum_hbm.at[pl.ds(tbase, _TBL)])


_hist = pl.kernel(
    _hist_body,
    out_type=(
        jax.ShapeDtypeStruct((_NW * _TBL,), jnp.float32),
        jax.ShapeDtypeStruct((_NW * _TBL,), jnp.float32),
    ),
    mesh=plsc.VectorSubcoreMesh(core_axis_name="c", subcore_axis_name="s"),
    compiler_params=pltpu.CompilerParams(needs_layout_passes=False),
    scratch_types=[
        pltpu.VMEM((_WORDS_PER_TILE,), jnp.float32),
        pltpu.VMEM((_WORDS_PER_TILE,), jnp.int32),
        pltpu.VMEM((_TBL,), jnp.float32),
        pltpu.VMEM((_TBL,), jnp.float32),
    ],
)


def _epi_body(cnt_ref, bsum_ref, acc_ref, out_ref):
    cnt = jnp.sum(cnt_ref[...], axis=0)
    bsum = jnp.sum(bsum_ref[...], axis=0)
    acc = acc_ref[...]
    ne = cnt > 0.0
    accn = jnp.where(ne, _MMT * acc + (1.0 - _MMT) * cnt, acc)
    contrib = jnp.where(ne, bsum / jnp.where(ne, accn, 1.0), 0.0)
    n = jnp.sum(ne.astype(jnp.float32), axis=1)
    csum = jnp.sum(contrib, axis=1)
    n = jnp.where(n > 0.0, n, 1.0)
    loss = jnp.sum(csum / n) * (1.0 / _CLASS)
    out_ref[...] = loss[None, None]


_epilogue = pl.pallas_call(
    _epi_body,
    out_shape=jax.ShapeDtypeStruct((1, 1), jnp.float32),
)


def kernel(pred, target, acc_sum):
    cnt, bsum = _hist(pred.reshape(-1), target.reshape(-1))
    loss2d = _epilogue(
        cnt.reshape(_NW, _CLASS, _BINS),
        bsum.reshape(_NW, _CLASS, _BINS),
        acc_sum,
    )
    return loss2d[0, 0]


# speedup vs baseline: 185.6867x; 1.0138x over previous
import functools

import jax
import jax.numpy as jnp
from jax import lax
from jax.experimental import pallas as pl
from jax.experimental.pallas import tpu as pltpu
from jax.experimental.pallas import tpu_sc as plsc

_BINS = 30
_MMT = 0.6
_BATCH = 16384
_CLASS = 64

_NC = 2
_NS = 16
_NW = _NC * _NS
_L = 16

_ROWS_PER_TILE = _BATCH // _NW
_WORDS_PER_TILE = _ROWS_PER_TILE * _CLASS
_TBL = _CLASS * _BINS


def _hist_body(pred_hbm, tgt_hbm, cnt_hbm, bsum_hbm, pred_v, tgt_v, cnt_v,
               bsum_v, sem_p0, sem_t0, sem_p1, sem_t1):
    wid = lax.axis_index("s") * _NC + lax.axis_index("c")
    base = wid * _WORDS_PER_TILE
    half = _WORDS_PER_TILE // 2

    cp = [
        pltpu.async_copy(pred_hbm.at[pl.ds(base, half)],
                         pred_v.at[pl.ds(0, half)], sem_p0),
        pltpu.async_copy(tgt_hbm.at[pl.ds(base, half)],
                         tgt_v.at[pl.ds(0, half)], sem_t0),
        pltpu.async_copy(pred_hbm.at[pl.ds(base + half, half)],
                         pred_v.at[pl.ds(half, half)], sem_p1),
        pltpu.async_copy(tgt_hbm.at[pl.ds(base + half, half)],
                         tgt_v.at[pl.ds(half, half)], sem_t1),
    ]

    zeros = jnp.zeros((_L,), jnp.float32)

    def zero_body(i, carry):
        cnt_v[pl.ds(i * _L, _L)] = zeros
        bsum_v[pl.ds(i * _L, _L)] = zeros
        return carry

    lax.fori_loop(0, _TBL // _L, zero_body, 0)

    lane = lax.iota(jnp.int32, _L)
    ones = jnp.ones((_L,), jnp.float32)
    base_idx = [(k * _L + lane) * _BINS for k in range(_CLASS // _L)]

    c = (2.211703e-05, 0.99901044, -0.48915684, 0.28330433, -0.13011941,
         0.030102625)

    _RPI = 2
    _KV = _RPI * (_CLASS // _L)

    def row_body(it, carry):
        off = it * (_RPI * _CLASS)
        ps = [pred_v[pl.ds(off + k * _L, _L)] for k in range(_KV)]
        ts = [tgt_v[pl.ds(off + k * _L, _L)] for k in range(_KV)]
        qs = [jnp.where(t > 0, -p, p) for p, t in zip(ps, ts)]
        us = [jnp.exp(-jnp.abs(p)) for p in ps]
        for k in range(_KV):
            q, u = qs[k], us[k]
            l1p = c[5]
            for j in range(4, -1, -1):
                l1p = l1p * u + c[j]
            bce = jnp.maximum(q, 0.0) + l1p
            r1 = 1.0 / (1.0 + u)
            g = jnp.where(q >= 0.0, r1, u * r1)
            b = jnp.minimum((g * float(_BINS)).astype(jnp.int32), _BINS - 1)
            idx = base_idx[k % (_CLASS // _L)] + b
            plsc.addupdate_scatter(cnt_v, [idx], ones)
            plsc.addupdate_scatter(bsum_v, [idx], bce)
        return carry

    half_iters = _ROWS_PER_TILE // _RPI // 2
    cp[0].wait()
    cp[1].wait()
    lax.fori_loop(0, half_iters, row_body, 0, unroll=2)
    cp[2].wait()
    cp[3].wait()
    lax.fori_loop(half_iters, 2 * half_iters, row_body, 0, unroll=2)

    tbase = wid * _TBL
    pltpu.sync_copy(cnt_v, cnt_hbm.at[pl.ds(tbase, _TBL)])
    pltpu.sync_copy(bsum_v, bsum_hbm.at[pl.ds(tbase, _TBL)])


_hist = pl.kernel(
    _hist_body,
    out_type=(
        jax.ShapeDtypeStruct((_NW * _TBL,), jnp.float32),
        jax.ShapeDtypeStruct((_NW * _TBL,), jnp.float32),
    ),
    mesh=plsc.VectorSubcoreMesh(core_axis_name="c", subcore_axis_name="s"),
    compiler_params=pltpu.CompilerParams(needs_layout_passes=False),
    scratch_types=[
        pltpu.VMEM((_WORDS_PER_TILE,), jnp.float32),
        pltpu.VMEM((_WORDS_PER_TILE,), jnp.int32),
        pltpu.VMEM((_TBL,), jnp.float32),
        pltpu.VMEM((_TBL,), jnp.float32),
        pltpu.SemaphoreType.DMA,
        pltpu.SemaphoreType.DMA,
        pltpu.SemaphoreType.DMA,
        pltpu.SemaphoreType.DMA,
    ],
)


def _epi_body(cnt_ref, bsum_ref, acc_ref, out_ref):
    cnt = jnp.sum(cnt_ref[...], axis=0)
    bsum = jnp.sum(bsum_ref[...], axis=0)
    acc = acc_ref[...]
    ne = cnt > 0.0
    accn = jnp.where(ne, _MMT * acc + (1.0 - _MMT) * cnt, acc)
    contrib = jnp.where(ne, bsum / jnp.where(ne, accn, 1.0), 0.0)
    n = jnp.sum(ne.astype(jnp.float32), axis=1)
    csum = jnp.sum(contrib, axis=1)
    n = jnp.where(n > 0.0, n, 1.0)
    loss = jnp.sum(csum / n) * (1.0 / _CLASS)
    out_ref[...] = loss[None, None]


_epilogue = pl.pallas_call(
    _epi_body,
    out_shape=jax.ShapeDtypeStruct((1, 1), jnp.float32),
)


def kernel(pred, target, acc_sum):
    cnt, bsum = _hist(pred.reshape(-1), target.reshape(-1))
    loss2d = _epilogue(
        cnt.reshape(_NW, _CLASS, _BINS),
        bsum.reshape(_NW, _CLASS, _BINS),
        acc_sum,
    )
    return loss2d[0, 0]


# speedup vs baseline: 239.0606x; 1.2874x over previous
import functools

import jax
import jax.numpy as jnp
from jax import lax
from jax.experimental import pallas as pl
from jax.experimental.pallas import tpu as pltpu
from jax.experimental.pallas import tpu_sc as plsc

_BINS = 30
_MMT = 0.6
_BATCH = 16384
_CLASS = 64

_NC = 2
_NS = 16
_NW = _NC * _NS
_L = 16

_ROWS_PER_TILE = _BATCH // _NW
_WORDS_PER_TILE = _ROWS_PER_TILE * _CLASS
_TBL = _CLASS * _BINS


_CH = 128
_NCHUNK = _ROWS_PER_TILE // _CH


def _hist_body(pred_hbm, tgt_hbm, tab_hbm, p0, p1, t0, t1, cnt_v,
               bsum_v, sp0, sp1, st0, st1):
    wid = lax.axis_index("s") * _NC + lax.axis_index("c")
    rbase = wid * _ROWS_PER_TILE
    pbufs, tbufs = [p0, p1], [t0, t1]
    psems, tsems = [sp0, sp1], [st0, st1]

    def start(ch):
        return (
            pltpu.async_copy(pred_hbm.at[pl.ds(rbase + ch * _CH, _CH), :],
                             pbufs[ch % 2], psems[ch % 2]),
            pltpu.async_copy(tgt_hbm.at[pl.ds(rbase + ch * _CH, _CH), :],
                             tbufs[ch % 2], tsems[ch % 2]),
        )

    cps = {0: start(0), 1: start(1)}

    zeros = jnp.zeros((_L,), jnp.float32)

    def zero_body(i, carry):
        cnt_v[pl.ds(i * _L, _L)] = zeros
        bsum_v[pl.ds(i * _L, _L)] = zeros
        return carry

    lax.fori_loop(0, _TBL // _L, zero_body, 0)

    lane = lax.iota(jnp.int32, _L)
    ones = jnp.ones((_L,), jnp.float32)
    base_idx = [(k * _L + lane) * _BINS for k in range(_CLASS // _L)]

    c = (2.211703e-05, 0.99901044, -0.48915684, 0.28330433, -0.13011941,
         0.030102625)

    _RPI = 2
    _KV = _RPI * (_CLASS // _L)

    def make_body(pv, tv):
        def row_body(it, carry):
            r0 = it * _RPI
            ps = [pv[r0 + k // 4, pl.ds((k % 4) * _L, _L)] for k in range(_KV)]
            ts = [tv[r0 + k // 4, pl.ds((k % 4) * _L, _L)] for k in range(_KV)]
            qs = [jnp.where(t > 0, -p, p) for p, t in zip(ps, ts)]
            us = [jnp.exp(-jnp.abs(p)) for p in ps]
            for k in range(_KV):
                q, u = qs[k], us[k]
                l1p = c[5]
                for j in range(4, -1, -1):
                    l1p = l1p * u + c[j]
                bce = jnp.maximum(q, 0.0) + l1p
                r1 = 1.0 / (1.0 + u)
                g = jnp.where(q >= 0.0, r1, u * r1)
                b = jnp.minimum((g * float(_BINS)).astype(jnp.int32), _BINS - 1)
                idx = base_idx[k % (_CLASS // _L)] + b
                plsc.addupdate_scatter(cnt_v, [idx], ones)
                plsc.addupdate_scatter(bsum_v, [idx], bce)
            return carry
        return row_body

    for ch in range(_NCHUNK):
        cp_p, cp_t = cps[ch]
        cp_p.wait()
        cp_t.wait()
        lax.fori_loop(0, _CH // _RPI, make_body(pbufs[ch % 2], tbufs[ch % 2]),
                      0, unroll=2)
        if ch + 2 < _NCHUNK:
            cps[ch + 2] = start(ch + 2)

    tbase = wid * _TBL
    pltpu.sync_copy(cnt_v, tab_hbm.at[pl.ds(tbase, _TBL)])
    pltpu.sync_copy(bsum_v, tab_hbm.at[pl.ds(_NW * _TBL + tbase, _TBL)])


_hist = pl.kernel(
    _hist_body,
    out_type=jax.ShapeDtypeStruct((2 * _NW * _TBL,), jnp.float32),
    mesh=plsc.VectorSubcoreMesh(core_axis_name="c", subcore_axis_name="s"),
    compiler_params=pltpu.CompilerParams(needs_layout_passes=False),
    scratch_types=[
        pltpu.VMEM((_CH, _CLASS), jnp.float32),
        pltpu.VMEM((_CH, _CLASS), jnp.float32),
        pltpu.VMEM((_CH, _CLASS), jnp.int32),
        pltpu.VMEM((_CH, _CLASS), jnp.int32),
        pltpu.VMEM((_TBL,), jnp.float32),
        pltpu.VMEM((_TBL,), jnp.float32),
        pltpu.SemaphoreType.DMA,
        pltpu.SemaphoreType.DMA,
        pltpu.SemaphoreType.DMA,
        pltpu.SemaphoreType.DMA,
    ],
)


def _epi_body(tab_ref, acc_ref, out_ref):
    tab = tab_ref[...]
    cnt = jnp.sum(tab[0], axis=0)
    bsum = jnp.sum(tab[1], axis=0)
    acc = acc_ref[...]
    ne = cnt > 0.0
    accn = jnp.where(ne, _MMT * acc + (1.0 - _MMT) * cnt, acc)
    contrib = jnp.where(ne, bsum / jnp.where(ne, accn, 1.0), 0.0)
    n = jnp.sum(ne.astype(jnp.float32), axis=1)
    csum = jnp.sum(contrib, axis=1)
    n = jnp.where(n > 0.0, n, 1.0)
    loss = jnp.sum(csum / n) * (1.0 / _CLASS)
    out_ref[...] = loss[None, None]


_epilogue = pl.pallas_call(
    _epi_body,
    out_shape=jax.ShapeDtypeStruct((1, 1), jnp.float32),
)


def kernel(pred, target, acc_sum):
    tab = _hist(pred, target)
    loss2d = _epilogue(tab.reshape(2, _NW, _CLASS, _BINS), acc_sum)
    return loss2d[0, 0]


# speedup vs baseline: 248.9717x; 1.0415x over previous
import functools

import jax
import jax.numpy as jnp
from jax import lax
from jax.experimental import pallas as pl
from jax.experimental.pallas import tpu as pltpu
from jax.experimental.pallas import tpu_sc as plsc

_BINS = 30
_MMT = 0.6
_BATCH = 16384
_CLASS = 64

_NC = 2
_NS = 16
_NW = _NC * _NS
_L = 16

_ROWS_PER_TILE = _BATCH // _NW
_PAD = 128


_CH = 128
_NCHUNK = _ROWS_PER_TILE // _CH


def _hist_body(pred_hbm, tgt_hbm, tab_hbm, p0, p1, t0, t1, cnt_v,
               bsum_v, sp0, sp1, st0, st1):
    wid = lax.axis_index("s") * _NC + lax.axis_index("c")
    rbase = wid * _ROWS_PER_TILE
    pbufs, tbufs = [p0, p1], [t0, t1]
    psems, tsems = [sp0, sp1], [st0, st1]

    def start(ch):
        return (
            pltpu.async_copy(pred_hbm.at[pl.ds(rbase + ch * _CH, _CH), :],
                             pbufs[ch % 2], psems[ch % 2]),
            pltpu.async_copy(tgt_hbm.at[pl.ds(rbase + ch * _CH, _CH), :],
                             tbufs[ch % 2], tsems[ch % 2]),
        )

    cps = {0: start(0), 1: start(1)}

    zeros = jnp.zeros((_L,), jnp.float32)

    def zero_body(r, carry):
        for j in range(_PAD // _L):
            cnt_v[r, pl.ds(j * _L, _L)] = zeros
            bsum_v[r, pl.ds(j * _L, _L)] = zeros
        return carry

    lax.fori_loop(0, _CLASS, zero_body, 0, unroll=2)

    lane = lax.iota(jnp.int32, _L)
    ones = jnp.ones((_L,), jnp.float32)
    cls_idx = [k * _L + lane for k in range(_CLASS // _L)]

    c = (2.211703e-05, 0.99901044, -0.48915684, 0.28330433, -0.13011941,
         0.030102625)

    _RPI = 2
    _KV = _RPI * (_CLASS // _L)

    def make_body(pv, tv):
        def row_body(it, carry):
            r0 = it * _RPI
            ps = [pv[r0 + k // 4, pl.ds((k % 4) * _L, _L)] for k in range(_KV)]
            ts = [tv[r0 + k // 4, pl.ds((k % 4) * _L, _L)] for k in range(_KV)]
            qs = [jnp.where(t > 0, -p, p) for p, t in zip(ps, ts)]
            us = [jnp.exp(-jnp.abs(p)) for p in ps]
            for k in range(_KV):
                q, u = qs[k], us[k]
                l1p = c[5]
                for j in range(4, -1, -1):
                    l1p = l1p * u + c[j]
                bce = jnp.maximum(q, 0.0) + l1p
                r1 = 1.0 / (1.0 + u)
                g = jnp.where(q >= 0.0, r1, u * r1)
                b = jnp.minimum((g * float(_BINS)).astype(jnp.int32), _BINS - 1)
                ci = cls_idx[k % (_CLASS // _L)]
                plsc.addupdate_scatter(cnt_v, [ci, b], ones)
                plsc.addupdate_scatter(bsum_v, [ci, b], bce)
            return carry
        return row_body

    for ch in range(_NCHUNK):
        cp_p, cp_t = cps[ch]
        cp_p.wait()
        cp_t.wait()
        lax.fori_loop(0, _CH // _RPI, make_body(pbufs[ch % 2], tbufs[ch % 2]),
                      0, unroll=4)
        if ch + 2 < _NCHUNK:
            cps[ch + 2] = start(ch + 2)

    pltpu.sync_copy(cnt_v, tab_hbm.at[0, wid])
    pltpu.sync_copy(bsum_v, tab_hbm.at[1, wid])


_hist = pl.kernel(
    _hist_body,
    out_type=jax.ShapeDtypeStruct((2, _NW, _CLASS, _PAD), jnp.float32),
    mesh=plsc.VectorSubcoreMesh(core_axis_name="c", subcore_axis_name="s"),
    compiler_params=pltpu.CompilerParams(needs_layout_passes=False),
    scratch_types=[
        pltpu.VMEM((_CH, _CLASS), jnp.float32),
        pltpu.VMEM((_CH, _CLASS), jnp.float32),
        pltpu.VMEM((_CH, _CLASS), jnp.int32),
        pltpu.VMEM((_CH, _CLASS), jnp.int32),
        pltpu.VMEM((_CLASS, _PAD), jnp.float32),
        pltpu.VMEM((_CLASS, _PAD), jnp.float32),
        pltpu.SemaphoreType.DMA,
        pltpu.SemaphoreType.DMA,
        pltpu.SemaphoreType.DMA,
        pltpu.SemaphoreType.DMA,
    ],
)


def _epi_body(tab_ref, acc_ref, out_ref):
    tab = tab_ref[...]
    cnt = jnp.sum(tab[0], axis=0)
    bsum = jnp.sum(tab[1], axis=0)
    acc = jnp.pad(acc_ref[...], ((0, 0), (0, _PAD - _BINS)))
    ne = cnt > 0.0
    accn = jnp.where(ne, _MMT * acc + (1.0 - _MMT) * cnt, acc)
    contrib = jnp.where(ne, bsum / jnp.where(ne, accn, 1.0), 0.0)
    n = jnp.sum(ne.astype(jnp.float32), axis=1)
    csum = jnp.sum(contrib, axis=1)
    n = jnp.where(n > 0.0, n, 1.0)
    loss = jnp.sum(csum / n) * (1.0 / _CLASS)
    out_ref[...] = loss[None, None]


_epilogue = pl.pallas_call(
    _epi_body,
    out_shape=jax.ShapeDtypeStruct((1, 1), jnp.float32),
)


def kernel(pred, target, acc_sum):
    tab = _hist(pred, target)
    loss2d = _epilogue(tab, acc_sum)
    return loss2d[0, 0]


# speedup vs baseline: 255.0027x; 1.0242x over previous
import functools

import jax
import jax.numpy as jnp
from jax import lax
from jax.experimental import pallas as pl
from jax.experimental.pallas import tpu as pltpu
from jax.experimental.pallas import tpu_sc as plsc

_BINS = 30
_MMT = 0.6
_BATCH = 16384
_CLASS = 64

_NC = 2
_NS = 16
_NW = _NC * _NS
_L = 16

_ROWS_PER_TILE = _BATCH // _NW
_PAD = 128


_CH = 128
_NCHUNK = _ROWS_PER_TILE // _CH


def _hist_body(pred_hbm, tgt_hbm, tab_hbm, p0, p1, t0, t1, cnt_v,
               bsum_v, sp0, sp1, st0, st1):
    wid = lax.axis_index("s") * _NC + lax.axis_index("c")
    rbase = wid * _ROWS_PER_TILE
    pbufs, tbufs = [p0, p1], [t0, t1]
    psems, tsems = [sp0, sp1], [st0, st1]

    def start(ch):
        return (
            pltpu.async_copy(pred_hbm.at[pl.ds(rbase + ch * _CH, _CH), :],
                             pbufs[ch % 2], psems[ch % 2]),
            pltpu.async_copy(tgt_hbm.at[pl.ds(rbase + ch * _CH, _CH), :],
                             tbufs[ch % 2], tsems[ch % 2]),
        )

    cps = {0: start(0), 1: start(1)}

    zeros = jnp.zeros((_L,), jnp.float32)

    def zero_body(r, carry):
        for j in range(2):
            cnt_v[r, pl.ds(j * _L, _L)] = zeros
            bsum_v[r, pl.ds(j * _L, _L)] = zeros
        return carry

    lax.fori_loop(0, _CLASS, zero_body, 0, unroll=4)

    lane = lax.iota(jnp.int32, _L)
    ones = jnp.ones((_L,), jnp.float32)
    cls_idx = [k * _L + lane for k in range(_CLASS // _L)]

    c = (2.211703e-05, 0.99901044, -0.48915684, 0.28330433, -0.13011941,
         0.030102625)

    _RPI = 4
    _KV = _RPI * (_CLASS // _L)

    def make_body(pv, tv):
        def row_body(it, carry):
            r0 = it * _RPI
            ps = [pv[r0 + k // 4, pl.ds((k % 4) * _L, _L)] for k in range(_KV)]
            ts = [tv[r0 + k // 4, pl.ds((k % 4) * _L, _L)] for k in range(_KV)]
            qs = [jnp.where(t > 0, -p, p) for p, t in zip(ps, ts)]
            us = [jnp.exp(-jnp.abs(p)) for p in ps]
            for k in range(_KV):
                q, u = qs[k], us[k]
                l1p = c[5]
                for j in range(4, -1, -1):
                    l1p = l1p * u + c[j]
                bce = jnp.maximum(q, 0.0) + l1p
                r1 = 1.0 / (1.0 + u)
                g = jnp.where(q >= 0.0, r1, u * r1)
                b = jnp.minimum((g * float(_BINS)).astype(jnp.int32), _BINS - 1)
                ci = cls_idx[k % (_CLASS // _L)]
                plsc.addupdate_scatter(cnt_v, [ci, b], ones)
                plsc.addupdate_scatter(bsum_v, [ci, b], bce)
            return carry
        return row_body

    for ch in range(_NCHUNK):
        cp_p, cp_t = cps[ch]
        cp_p.wait()
        cp_t.wait()
        lax.fori_loop(0, _CH // _RPI, make_body(pbufs[ch % 2], tbufs[ch % 2]),
                      0, unroll=2)
        if ch + 2 < _NCHUNK:
            cps[ch + 2] = start(ch + 2)

    pltpu.sync_copy(cnt_v, tab_hbm.at[0, wid])
    pltpu.sync_copy(bsum_v, tab_hbm.at[1, wid])


_hist = pl.kernel(
    _hist_body,
    out_type=jax.ShapeDtypeStruct((2, _NW, _CLASS, _PAD), jnp.float32),
    mesh=plsc.VectorSubcoreMesh(core_axis_name="c", subcore_axis_name="s"),
    compiler_params=pltpu.CompilerParams(needs_layout_passes=False),
    scratch_types=[
        pltpu.VMEM((_CH, _CLASS), jnp.float32),
        pltpu.VMEM((_CH, _CLASS), jnp.float32),
        pltpu.VMEM((_CH, _CLASS), jnp.int32),
        pltpu.VMEM((_CH, _CLASS), jnp.int32),
        pltpu.VMEM((_CLASS, _PAD), jnp.float32),
        pltpu.VMEM((_CLASS, _PAD), jnp.float32),
        pltpu.SemaphoreType.DMA,
        pltpu.SemaphoreType.DMA,
        pltpu.SemaphoreType.DMA,
        pltpu.SemaphoreType.DMA,
    ],
)


def _epi_body(tab_ref, acc_ref, out_ref):
    tab = tab_ref[...]
    cnt = jnp.sum(tab[0], axis=0)[:, :_BINS]
    bsum = jnp.sum(tab[1], axis=0)[:, :_BINS]
    acc = acc_ref[...]
    ne = cnt > 0.0
    accn = jnp.where(ne, _MMT * acc + (1.0 - _MMT) * cnt, acc)
    contrib = jnp.where(ne, bsum / jnp.where(ne, accn, 1.0), 0.0)
    n = jnp.sum(ne.astype(jnp.float32), axis=1)
    csum = jnp.sum(contrib, axis=1)
    n = jnp.where(n > 0.0, n, 1.0)
    loss = jnp.sum(csum / n) * (1.0 / _CLASS)
    out_ref[...] = loss[None, None]


_epilogue = pl.pallas_call(
    _epi_body,
    out_shape=jax.ShapeDtypeStruct((1, 1), jnp.float32),
)


def kernel(pred, target, acc_sum):
    tab = _hist(pred, target)
    loss2d = _epilogue(tab, acc_sum)
    return loss2d[0, 0]
